# baseline (speedup 1.0000x reference)
import functools

import jax
import jax.numpy as jnp
from jax import lax
from jax.experimental import pallas as pl
from jax.experimental.pallas import tpu as pltpu
from jax.experimental.pallas import tpu_sc as plsc

D_MODEL = 64
SCALE = 8.0
NUM_CORES = 2
NUM_SUBCORES = 16
NW = NUM_CORES * NUM_SUBCORES
CHUNK = 128
NBUF = 6


def _retile_table(emb_weight):
    V = emb_weight.shape[0]
    embT = emb_weight.T
    BLK = 4096
    grid = -(-V // (2 * BLK))
    H = grid * BLK

    def body(a_ref, b_ref, out_ref):
        out_ref[:, 0:D_MODEL] = a_ref[...].T
        out_ref[:, D_MODEL:2 * D_MODEL] = b_ref[...].T

    paired = pl.pallas_call(
        body,
        grid=(grid,),
        in_specs=[
            pl.BlockSpec((D_MODEL, BLK), lambda g: (0, 2 * g)),
            pl.BlockSpec(
                (D_MODEL, BLK),
                lambda g: (0, jnp.minimum(2 * g + 1, (V - 1) // BLK)),
            ),
        ],
        out_specs=pl.BlockSpec((BLK, 2 * D_MODEL), lambda g: (g, 0)),
        out_shape=jax.ShapeDtypeStruct((H, 2 * D_MODEL), jnp.float32),
    )(embT, embT)
    return paired.reshape(2 * H, D_MODEL)


def kernel(x, emb_weight):
    B, T = x.shape
    N = B * T
    assert B == NW * CHUNK * (B // (NW * CHUNK))
    WB = B // NW
    assert WB == CHUNK

    xr = (x & ~8191) + 2 * (x & 4095) + ((x >> 12) & 1)
    idx3d = xr.astype(jnp.int32).T.reshape(T, NW, CHUNK)
    table_lin = _retile_table(emb_weight)

    DB = D_MODEL // 8

    mesh = plsc.VectorSubcoreMesh(
        core_axis_name="c",
        subcore_axis_name="s",
        num_cores=NUM_CORES,
        num_subcores=NUM_SUBCORES,
    )

    @functools.partial(
        pl.kernel,
        out_type=jax.ShapeDtypeStruct((T, DB, NW, 8, CHUNK), jnp.float32),
        mesh=mesh,
        scratch_types=[
            pltpu.VMEM((T, CHUNK), jnp.int32),
            pltpu.VMEM((NBUF, CHUNK, D_MODEL), jnp.float32),
            pltpu.VMEM((NBUF, DB, 8, CHUNK), jnp.float32),
            [pltpu.SemaphoreType.DMA] * NBUF,
            [pltpu.SemaphoreType.DMA] * NBUF,
        ],
        compiler_params=pltpu.CompilerParams(
            use_tc_tiling_on_sc=False, needs_layout_passes=False
        ),
    )
    def emb_kernel(idx_hbm, table_hbm, out_hbm, idx_v, bufs, bufts, gsems,
                   osems):
        wid = lax.axis_index("s") * NUM_CORES + lax.axis_index("c")
        pltpu.sync_copy(idx_hbm.at[:, wid], idx_v)

        def start_gather(t, b):
            pltpu.async_copy(table_hbm.at[idx_v.at[t]], bufs.at[b], gsems[b])

        def wait_gather(b):
            pltpu.make_async_copy(table_hbm.at[idx_v.at[0]], bufs.at[b],
                                  gsems[b]).wait()

        def start_out(t, b):
            pltpu.async_copy(bufts.at[b], out_hbm.at[t, :, wid], osems[b])

        def wait_out(b):
            pltpu.make_async_copy(bufts.at[b], out_hbm.at[0, :, wid],
                                  osems[b]).wait()

        def xscale(b):
            kidx = lax.iota(jnp.int32, 16)
            rots = [(kidx + j) & 15 for j in range(16)]
            rot7s = [r & 7 for r in rots]

            def blk(i, carry):
                rows = kidx + (i >> 2) * 16
                d0 = (i & 3) * 16
                for j in range(16):
                    d = rots[j] + d0
                    v = plsc.load_gather(bufs.at[b], [rows, d])
                    plsc.store_scatter(bufts.at[b], [d >> 3, rot7s[j], rows],
                                       v * SCALE)
                return carry

            lax.fori_loop(0, (CHUNK // 16) * (D_MODEL // 16), blk, 0)

        K = NBUF // 2

        for t in range(K):
            start_gather(t, t)

        for b in range(NBUF):
            t = b
            wait_gather(b)
            xscale(b)
            start_out(t, b)
            bg = (b + K) % NBUF
            if b >= K:
                wait_out(bg)
            start_gather(t + K, bg)

        n_waves = T // NBUF

        def wave(o, carry):
            for b in range(NBUF):
                t = o * NBUF + b
                wait_gather(b)
                xscale(b)
                start_out(t, b)
                bg = (b + K) % NBUF

                @pl.when(t + K < T)
                def _():
                    wait_out(bg)
                    start_gather(t + K, bg)

            return carry

        lax.fori_loop(1, n_waves, wave, 0)

        for b in range(T % NBUF):
            t = n_waves * NBUF + b
            wait_gather(b)
            xscale(b)
            start_out(t, b)

        for b in range(NBUF):
            wait_out(b)

    out5 = emb_kernel(idx3d, table_lin)
    return out5.transpose(2, 4, 0, 1, 3).reshape(B, T, D_MODEL)


# speedup vs baseline: 1.2995x; 1.2995x over previous
import functools

import jax
import jax.numpy as jnp
from jax import lax
from jax.experimental import pallas as pl
from jax.experimental.pallas import tpu as pltpu
from jax.experimental.pallas import tpu_sc as plsc

D_MODEL = 64
SCALE = 8.0
NUM_CORES = 2
NUM_SUBCORES = 16
NW = NUM_CORES * NUM_SUBCORES
CHUNK = 128
NBUF = 6


def _retile_table(emb_weight):
    V = emb_weight.shape[0]
    embT = emb_weight.T
    BLK = 4096
    grid = -(-V // (2 * BLK))
    H = grid * BLK

    def body(a_ref, b_ref, out_ref):
        out_ref[:, 0:D_MODEL] = a_ref[...].T
        out_ref[:, D_MODEL:2 * D_MODEL] = b_ref[...].T

    paired = pl.pallas_call(
        body,
        grid=(grid,),
        in_specs=[
            pl.BlockSpec((D_MODEL, BLK), lambda g: (0, 2 * g)),
            pl.BlockSpec(
                (D_MODEL, BLK),
                lambda g: (0, jnp.minimum(2 * g + 1, (V - 1) // BLK)),
            ),
        ],
        out_specs=pl.BlockSpec((BLK, 2 * D_MODEL), lambda g: (g, 0)),
        out_shape=jax.ShapeDtypeStruct((H, 2 * D_MODEL), jnp.float32),
    )(embT, embT)
    return paired.reshape(2 * H, D_MODEL)


def kernel(x, emb_weight):
    B, T = x.shape
    N = B * T
    assert B == NW * CHUNK * (B // (NW * CHUNK))
    WB = B // NW
    assert WB == CHUNK

    xr = (x & ~8191) + 2 * (x & 4095) + ((x >> 12) & 1)
    idx3d = xr.astype(jnp.int32).T.reshape(T, NW, CHUNK)
    table_lin = _retile_table(emb_weight)

    DB = D_MODEL // 8

    mesh = plsc.VectorSubcoreMesh(
        core_axis_name="c",
        subcore_axis_name="s",
        num_cores=NUM_CORES,
        num_subcores=NUM_SUBCORES,
    )

    @functools.partial(
        pl.kernel,
        out_type=jax.ShapeDtypeStruct((T, DB, NW, 8, CHUNK), jnp.float32),
        mesh=mesh,
        scratch_types=[
            pltpu.VMEM((T, CHUNK), jnp.int32),
            pltpu.VMEM((NBUF, CHUNK, D_MODEL), jnp.float32),
            pltpu.VMEM((NBUF, DB, 8, CHUNK), jnp.float32),
            [pltpu.SemaphoreType.DMA] * NBUF,
            [pltpu.SemaphoreType.DMA] * NBUF,
        ],
        compiler_params=pltpu.CompilerParams(
            use_tc_tiling_on_sc=False, needs_layout_passes=False
        ),
    )
    def emb_kernel(idx_hbm, table_hbm, out_hbm, idx_v, bufs, bufts, gsems,
                   osems):
        wid = lax.axis_index("s") * NUM_CORES + lax.axis_index("c")
        pltpu.sync_copy(idx_hbm.at[:, wid], idx_v)

        def start_gather(t, b):
            pltpu.async_copy(table_hbm.at[idx_v.at[t]], bufs.at[b], gsems[b])

        def wait_gather(b):
            pltpu.make_async_copy(table_hbm.at[idx_v.at[0]], bufs.at[b],
                                  gsems[b]).wait()

        def start_out(t, b):
            pltpu.async_copy(bufts.at[b], out_hbm.at[t, :, wid], osems[b])

        def wait_out(b):
            pltpu.make_async_copy(bufts.at[b], out_hbm.at[0, :, wid],
                                  osems[b]).wait()

        def xscale(b):
            kidx = lax.iota(jnp.int32, 16)
            rots = [(kidx + j) & 15 for j in range(16)]
            rot7s = [r & 7 for r in rots]

            @plsc.parallel_loop(0, (CHUNK // 16) * (D_MODEL // 16))
            def blk(i):
                rows = kidx + (i >> 2) * 16
                d0 = (i & 3) * 16
                for j in range(16):
                    d = rots[j] + d0
                    v = plsc.load_gather(bufs.at[b], [rows, d])
                    plsc.store_scatter(bufts.at[b], [d >> 3, rot7s[j], rows],
                                       v * SCALE)

        K = NBUF // 2

        for t in range(K):
            start_gather(t, t)

        for b in range(NBUF):
            t = b
            wait_gather(b)
            xscale(b)
            start_out(t, b)
            bg = (b + K) % NBUF
            if b >= K:
                wait_out(bg)
            start_gather(t + K, bg)

        n_waves = T // NBUF

        def wave(o, carry):
            for b in range(NBUF):
                t = o * NBUF + b
                wait_gather(b)
                xscale(b)
                start_out(t, b)
                bg = (b + K) % NBUF

                @pl.when(t + K < T)
                def _():
                    wait_out(bg)
                    start_gather(t + K, bg)

            return carry

        lax.fori_loop(1, n_waves, wave, 0)

        for b in range(T % NBUF):
            t = n_waves * NBUF + b
            wait_gather(b)
            xscale(b)
            start_out(t, b)

        for b in range(NBUF):
            wait_out(b)

    out5 = emb_kernel(idx3d, table_lin)
    return out5.transpose(2, 4, 0, 1, 3).reshape(B, T, D_MODEL)
